# baseline (speedup 1.0000x reference)
import functools

import jax
import jax.numpy as jnp
from jax import lax
from jax.experimental import pallas as pl
from jax.experimental.pallas import tpu as pltpu
from jax.experimental.pallas import tpu_sc as plsc

B = 16384
D = 128
NC = 2
NS = 16
NW = NC * NS
B_PER_W = B // NW
CHUNK = 128
N_CHUNKS = B_PER_W // CHUNK
N_JOBS = 3 * N_CHUNKS
GPS = 2
N_SJ = N_JOBS // GPS
NB = 3
N_REL = 1000
REL_PER_TILE = 64


def _gather3_body(emb_hbm, kgg_hbm, rel_hbm, scg_i_hbm, rel_i_hbm,
                  kgg_i_hbm, scg_out, kgg_out, rel_out,
                  idx_v, stage_v, rel_sh, *rest):
    rows = rest[:NB]
    isem = rest[NB]
    stsem = rest[NB + 1]
    gsems = rest[NB + 2:2 * NB + 2]
    ssems = rest[2 * NB + 2:]
    cid = lax.axis_index("c")
    sid = lax.axis_index("s")
    wid = sid * NC + cid
    base = wid * B_PER_W
    tables = (
        (emb_hbm, scg_i_hbm, scg_out),
        (kgg_hbm, kgg_i_hbm, kgg_out),
        (rel_sh, rel_i_hbm, rel_out),
    )
    jobs = []
    for h in range(N_CHUNKS // GPS):
        for t, (tab, ids2d, out) in enumerate(tables):
            for g in range(GPS):
                j = h * GPS + g
                jobs.append((tab, out, base + j * CHUNK, t * N_CHUNKS + j))

    stage_off = jnp.minimum(sid * REL_PER_TILE, N_REL - REL_PER_TILE)
    stage_in = pltpu.async_copy(
        rel_hbm.at[pl.ds(stage_off, REL_PER_TILE)], stage_v, stsem)
    idx_copies = []
    for t, (tab, ids2d, out) in enumerate(tables):
        idx_copies.append(pltpu.async_copy(
            ids2d.at[pl.ds(wid * N_CHUNKS, N_CHUNKS)],
            idx_v.at[pl.ds(t * N_CHUNKS, N_CHUNKS)], isem))
    for c in idx_copies:
        c.wait()

    gather_copies = [None] * N_JOBS
    store_copies = [None] * N_SJ

    def fire_gathers(sj):
        b = sj % NB
        for g in range(GPS):
            i = sj * GPS + g
            tab, out, off, irow = jobs[i]
            gather_copies[i] = pltpu.async_copy(
                tab.at[idx_v.at[irow]], rows[b].at[pl.ds(g * CHUNK, CHUNK)],
                gsems[b])

    def fire_store(sj):
        b = sj % NB
        tab, out, off, irow = jobs[sj * GPS]
        for g in range(GPS):
            gather_copies[sj * GPS + g].wait()
        store_copies[sj] = pltpu.async_copy(
            rows[b], out.at[pl.ds(off, GPS * CHUNK)], ssems[b])

    fire_gathers(0)
    fire_gathers(1)

    stage_in.wait()
    pltpu.async_copy(
        stage_v, rel_sh.at[pl.ds(stage_off, REL_PER_TILE)],
        stsem).wait()
    plsc.subcore_barrier()

    fire_gathers(2)
    for sj in range(NB):
        fire_store(sj)
    for sj in range(NB, N_SJ):
        store_copies[sj - NB].wait()
        fire_gathers(sj)
    for sj in range(NB, N_SJ):
        fire_store(sj)
    for sj in range(N_SJ - NB, N_SJ):
        store_copies[sj].wait()


@jax.jit
def _gather3(embedding, kgg_table, rel_table, scg_ids, relation_ids, kgg_ids):
    mesh = plsc.VectorSubcoreMesh(core_axis_name="c", subcore_axis_name="s")
    f = functools.partial(
        pl.kernel,
        mesh=mesh,
        out_type=(
            jax.ShapeDtypeStruct((B, D), jnp.float32),
            jax.ShapeDtypeStruct((B, D), jnp.float32),
            jax.ShapeDtypeStruct((B, D), jnp.float32),
        ),
        scratch_types=(
            [pltpu.VMEM((N_JOBS, CHUNK), jnp.int32),
             pltpu.VMEM((REL_PER_TILE, D), jnp.float32),
             pltpu.VMEM_SHARED((N_REL, D), jnp.float32)]
            + [pltpu.VMEM((GPS * CHUNK, D), jnp.float32) for _ in range(NB)]
            + [pltpu.SemaphoreType.DMA for _ in range(2 * NB + 2)]
        ),
    )(_gather3_body)
    return f(embedding, kgg_table, rel_table,
             scg_ids.reshape(B // CHUNK, CHUNK),
             relation_ids.reshape(B // CHUNK, CHUNK),
             kgg_ids.reshape(B // CHUNK, CHUNK))


def kernel(embedding, kgg_table, rel_table, scg_ids, relation_ids, kgg_ids):
    scg_ids = scg_ids.astype(jnp.int32)
    relation_ids = relation_ids.astype(jnp.int32)
    kgg_ids = kgg_ids.astype(jnp.int32)
    return _gather3(embedding, kgg_table, rel_table,
                    scg_ids, relation_ids, kgg_ids)


# speedup vs baseline: 1.0093x; 1.0093x over previous
import functools

import jax
import jax.numpy as jnp
from jax import lax
from jax.experimental import pallas as pl
from jax.experimental.pallas import tpu as pltpu
from jax.experimental.pallas import tpu_sc as plsc

B = 16384
D = 128
NC = 2
NS = 16
NW = NC * NS
B_PER_W = B // NW
CHUNK = 128
N_CHUNKS = B_PER_W // CHUNK
N_JOBS = 3 * N_CHUNKS
GPS = 2
N_SJ = N_JOBS // GPS
NB = 3
N_REL = 1000
REL_PER_TILE = 64


def _gather3_body(emb_hbm, kgg_hbm, rel_hbm, scg_i_hbm, rel_i_hbm,
                  kgg_i_hbm, scg_out, kgg_out, rel_out,
                  idx_v, stage_v, rel_sh, *rest):
    rows = rest[:NB]
    isem = rest[NB]
    stsem = rest[NB + 1]
    gsems = rest[NB + 2:2 * NB + 2]
    ssems = rest[2 * NB + 2:]
    cid = lax.axis_index("c")
    sid = lax.axis_index("s")
    wid = sid * NC + cid
    base = wid * B_PER_W
    tables = (
        (emb_hbm, scg_i_hbm, scg_out),
        (kgg_hbm, kgg_i_hbm, kgg_out),
        (rel_sh, rel_i_hbm, rel_out),
    )
    jobs = []
    for t, (tab, ids2d, out) in enumerate(tables):
        for j in range(N_CHUNKS):
            jobs.append((tab, out, base + j * CHUNK, t * N_CHUNKS + j))

    stage_off = jnp.minimum(sid * REL_PER_TILE, N_REL - REL_PER_TILE)
    stage_in = pltpu.async_copy(
        rel_hbm.at[pl.ds(stage_off, REL_PER_TILE)], stage_v, stsem)
    idx_copies = []
    for t, (tab, ids2d, out) in enumerate(tables):
        idx_copies.append(pltpu.async_copy(
            ids2d.at[pl.ds(wid * N_CHUNKS, N_CHUNKS)],
            idx_v.at[pl.ds(t * N_CHUNKS, N_CHUNKS)], isem))
    for c in idx_copies:
        c.wait()

    gather_copies = [None] * N_JOBS
    store_copies = [None] * N_SJ

    def fire_gathers(sj):
        b = sj % NB
        for g in range(GPS):
            i = sj * GPS + g
            tab, out, off, irow = jobs[i]
            gather_copies[i] = pltpu.async_copy(
                tab.at[idx_v.at[irow]], rows[b].at[pl.ds(g * CHUNK, CHUNK)],
                gsems[b])

    def fire_store(sj):
        b = sj % NB
        tab, out, off, irow = jobs[sj * GPS]
        for g in range(GPS):
            gather_copies[sj * GPS + g].wait()
        store_copies[sj] = pltpu.async_copy(
            rows[b], out.at[pl.ds(off, GPS * CHUNK)], ssems[b])

    for sj in range(NB):
        fire_gathers(sj)

    stage_in.wait()
    pltpu.async_copy(
        stage_v, rel_sh.at[pl.ds(stage_off, REL_PER_TILE)],
        stsem).wait()

    for sj in range(NB):
        fire_store(sj)
    store_copies[0].wait()
    fire_gathers(3)

    plsc.subcore_barrier()

    for sj in range(NB + 1, N_SJ):
        store_copies[sj - NB].wait()
        fire_gathers(sj)
    for sj in range(NB, N_SJ):
        fire_store(sj)
    for sj in range(N_SJ - NB, N_SJ):
        store_copies[sj].wait()


@jax.jit
def _gather3(embedding, kgg_table, rel_table, scg_ids, relation_ids, kgg_ids):
    mesh = plsc.VectorSubcoreMesh(core_axis_name="c", subcore_axis_name="s")
    f = functools.partial(
        pl.kernel,
        mesh=mesh,
        out_type=(
            jax.ShapeDtypeStruct((B, D), jnp.float32),
            jax.ShapeDtypeStruct((B, D), jnp.float32),
            jax.ShapeDtypeStruct((B, D), jnp.float32),
        ),
        scratch_types=(
            [pltpu.VMEM((N_JOBS, CHUNK), jnp.int32),
             pltpu.VMEM((REL_PER_TILE, D), jnp.float32),
             pltpu.VMEM_SHARED((N_REL, D), jnp.float32)]
            + [pltpu.VMEM((GPS * CHUNK, D), jnp.float32) for _ in range(NB)]
            + [pltpu.SemaphoreType.DMA for _ in range(2 * NB + 2)]
        ),
    )(_gather3_body)
    return f(embedding, kgg_table, rel_table,
             scg_ids.reshape(B // CHUNK, CHUNK),
             relation_ids.reshape(B // CHUNK, CHUNK),
             kgg_ids.reshape(B // CHUNK, CHUNK))


def kernel(embedding, kgg_table, rel_table, scg_ids, relation_ids, kgg_ids):
    scg_ids = scg_ids.astype(jnp.int32)
    relation_ids = relation_ids.astype(jnp.int32)
    kgg_ids = kgg_ids.astype(jnp.int32)
    return _gather3(embedding, kgg_table, rel_table,
                    scg_ids, relation_ids, kgg_ids)


# speedup vs baseline: 1.0277x; 1.0182x over previous
import functools

import jax
import jax.numpy as jnp
from jax import lax
from jax.experimental import pallas as pl
from jax.experimental.pallas import tpu as pltpu
from jax.experimental.pallas import tpu_sc as plsc

B = 16384
D = 128
NC = 2
NS = 16
NW = NC * NS
B_PER_W = B // NW
CHUNK = 128
N_CHUNKS = B_PER_W // CHUNK
N_JOBS = 3 * N_CHUNKS
GPS = 1
N_SJ = N_JOBS // GPS
NB = 6
N_REL = 1000
REL_PER_TILE = 64


def _gather3_body(emb_hbm, kgg_hbm, rel_hbm, scg_i_hbm, rel_i_hbm,
                  kgg_i_hbm, scg_out, kgg_out, rel_out,
                  idx_v, stage_v, rel_sh, *rest):
    rows = rest[:NB]
    isem = rest[NB]
    stsem = rest[NB + 1]
    gsems = rest[NB + 2:2 * NB + 2]
    ssems = rest[2 * NB + 2:]
    cid = lax.axis_index("c")
    sid = lax.axis_index("s")
    wid = sid * NC + cid
    base = wid * B_PER_W
    tables = (
        (emb_hbm, scg_i_hbm, scg_out),
        (kgg_hbm, kgg_i_hbm, kgg_out),
        (rel_sh, rel_i_hbm, rel_out),
    )
    jobs = []
    for t, (tab, ids2d, out) in enumerate(tables):
        for j in range(N_CHUNKS):
            jobs.append((tab, out, base + j * CHUNK, t * N_CHUNKS + j))

    stage_off = jnp.minimum(sid * REL_PER_TILE, N_REL - REL_PER_TILE)
    stage_in = pltpu.async_copy(
        rel_hbm.at[pl.ds(stage_off, REL_PER_TILE)], stage_v, stsem)
    idx_copies = []
    for t, (tab, ids2d, out) in enumerate(tables):
        idx_copies.append(pltpu.async_copy(
            ids2d.at[pl.ds(wid * N_CHUNKS, N_CHUNKS)],
            idx_v.at[pl.ds(t * N_CHUNKS, N_CHUNKS)], isem))
    for c in idx_copies:
        c.wait()

    gather_copies = [None] * N_JOBS
    store_copies = [None] * N_SJ

    def fire_gathers(sj):
        b = sj % NB
        for g in range(GPS):
            i = sj * GPS + g
            tab, out, off, irow = jobs[i]
            gather_copies[i] = pltpu.async_copy(
                tab.at[idx_v.at[irow]], rows[b].at[pl.ds(g * CHUNK, CHUNK)],
                gsems[b])

    def fire_store(sj):
        b = sj % NB
        tab, out, off, irow = jobs[sj * GPS]
        for g in range(GPS):
            gather_copies[sj * GPS + g].wait()
        store_copies[sj] = pltpu.async_copy(
            rows[b], out.at[pl.ds(off, GPS * CHUNK)], ssems[b])

    for sj in range(NB):
        fire_gathers(sj)

    stage_in.wait()
    pltpu.async_copy(
        stage_v, rel_sh.at[pl.ds(stage_off, REL_PER_TILE)],
        stsem).wait()

    for sj in range(NB):
        fire_store(sj)
    rel_sj_start = 2 * N_CHUNKS // GPS
    for sj in range(NB, N_SJ):
        store_copies[sj - NB].wait()
        if sj == rel_sj_start:
            plsc.subcore_barrier()
        fire_gathers(sj)
    for sj in range(NB, N_SJ):
        fire_store(sj)
    for sj in range(N_SJ - NB, N_SJ):
        store_copies[sj].wait()


@jax.jit
def _gather3(embedding, kgg_table, rel_table, scg_ids, relation_ids, kgg_ids):
    mesh = plsc.VectorSubcoreMesh(core_axis_name="c", subcore_axis_name="s")
    f = functools.partial(
        pl.kernel,
        mesh=mesh,
        out_type=(
            jax.ShapeDtypeStruct((B, D), jnp.float32),
            jax.ShapeDtypeStruct((B, D), jnp.float32),
            jax.ShapeDtypeStruct((B, D), jnp.float32),
        ),
        scratch_types=(
            [pltpu.VMEM((N_JOBS, CHUNK), jnp.int32),
             pltpu.VMEM((REL_PER_TILE, D), jnp.float32),
             pltpu.VMEM_SHARED((N_REL, D), jnp.float32)]
            + [pltpu.VMEM((GPS * CHUNK, D), jnp.float32) for _ in range(NB)]
            + [pltpu.SemaphoreType.DMA for _ in range(2 * NB + 2)]
        ),
    )(_gather3_body)
    return f(embedding, kgg_table, rel_table,
             scg_ids.reshape(B // CHUNK, CHUNK),
             relation_ids.reshape(B // CHUNK, CHUNK),
             kgg_ids.reshape(B // CHUNK, CHUNK))


def kernel(embedding, kgg_table, rel_table, scg_ids, relation_ids, kgg_ids):
    scg_ids = scg_ids.astype(jnp.int32)
    relation_ids = relation_ids.astype(jnp.int32)
    kgg_ids = kgg_ids.astype(jnp.int32)
    return _gather3(embedding, kgg_table, rel_table,
                    scg_ids, relation_ids, kgg_ids)
